# baseline (speedup 1.0000x reference)
import functools

import jax
import jax.numpy as jnp
from jax import lax
from jax.experimental import pallas as pl
from jax.experimental.pallas import tpu as pltpu
from jax.experimental.pallas import tpu_sc as plsc

VOCAB = 1000000
EMBED = 64
SEQ = 200
BATCH = 4096

NC, NS = 2, 16
NW = NC * NS
N = BATCH * SEQ
CHUNK = 128
ROWS_PER_W = N // NW
CHUNKS_PER_W = ROWS_PER_W // CHUNK


def _sc_embed(seq2, lab2, tok2, pos2, seg2):
  mesh = plsc.VectorSubcoreMesh(core_axis_name="c", subcore_axis_name="s")

  @functools.partial(
      pl.kernel,
      mesh=mesh,
      compiler_params=pltpu.CompilerParams(use_tc_tiling_on_sc=True,
                                           needs_layout_passes=False),
      out_type=jax.ShapeDtypeStruct((N, EMBED), jnp.float32),
      scratch_types=[
          pltpu.VMEM((CHUNKS_PER_W, CHUNK), jnp.int32),
          pltpu.VMEM((CHUNKS_PER_W, CHUNK), jnp.int32),
          pltpu.VMEM((SEQ * EMBED // 128, 128), jnp.float32),
          pltpu.VMEM((2, 128), jnp.float32),
          pltpu.VMEM((CHUNK, 128), jnp.float32),
          pltpu.VMEM((CHUNK, 128), jnp.float32),
          pltpu.VMEM((CHUNK, EMBED), jnp.float32),
          pltpu.VMEM((CHUNK, EMBED), jnp.float32),
          pltpu.SemaphoreType.DMA,
          pltpu.SemaphoreType.DMA,
          pltpu.SemaphoreType.DMA,
          pltpu.SemaphoreType.DMA,
      ],
  )
  def k(seq_hbm, lab_hbm, tok_hbm, pos_hbm, seg_hbm, out_hbm,
        idx_v, meta_v, pos_v, seg_v, tok0, tok1, out0, out1,
        gs0, gs1, ss0, ss1):
    wid = lax.axis_index("s") * NC + lax.axis_index("c")
    r0 = pl.multiple_of(wid * CHUNKS_PER_W, 8)
    flat0 = pl.multiple_of(wid * ROWS_PER_W, 1024)

    pltpu.sync_copy(seq_hbm.at[pl.ds(r0, CHUNKS_PER_W)], idx_v)
    pltpu.sync_copy(lab_hbm.at[pl.ds(r0, CHUNKS_PER_W)], meta_v)
    pltpu.sync_copy(pos_hbm, pos_v)
    pltpu.sync_copy(seg_hbm, seg_v)

    iota = lax.iota(jnp.int32, 16)

    @plsc.parallel_loop(0, CHUNKS_PER_W, unroll=2)
    def fuse(r):
      base = flat0 + r * CHUNK
      for i in range(CHUNK // 16):
        sl = pl.ds(i * 16, 16)
        pos = (base + i * 16 + iota) % SEQ
        v = idx_v[r, sl]
        meta_v[r, sl] = (pos * 4 + meta_v[r, sl]) * 2 + (v & 1)
        idx_v[r, sl] = v >> 1

    def g_start(j, tok_b, sem):
      pltpu.async_copy(tok_hbm.at[idx_v.at[j, pl.ds(0, CHUNK // 2)]],
                       tok_b.at[pl.ds(0, CHUNK // 2)], sem)
      pltpu.async_copy(tok_hbm.at[idx_v.at[j, pl.ds(CHUNK // 2, CHUNK // 2)]],
                       tok_b.at[pl.ds(CHUNK // 2, CHUNK // 2)], sem)

    def g_wait(j, tok_b, sem):
      pltpu.make_async_copy(tok_hbm.at[idx_v.at[j, pl.ds(0, CHUNK // 2)]],
                            tok_b.at[pl.ds(0, CHUNK // 2)], sem).wait()
      pltpu.make_async_copy(tok_hbm.at[idx_v.at[j, pl.ds(CHUNK // 2, CHUNK // 2)]],
                            tok_b.at[pl.ds(CHUNK // 2, CHUNK // 2)], sem).wait()

    def s_start(j, out_b, sem):
      off = pl.multiple_of(flat0 + j * CHUNK, 8)
      pltpu.async_copy(out_b, out_hbm.at[pl.ds(off, CHUNK)], sem)

    def s_wait(j, out_b, sem):
      off = pl.multiple_of(flat0 + j * CHUNK, 8)
      pltpu.make_async_copy(out_b, out_hbm.at[pl.ds(off, CHUNK)], sem).wait()

    def process(j, tok_b, out_b):
      @plsc.parallel_loop(0, CHUNK // 16, unroll=4)
      def group(g):
        mvec = meta_v[j, pl.ds(g * 16, 16)]
        for jj in range(16):
          m = mvec[jj]
          h64 = (m & 1) * EMBED
          mm = m >> 1
          posf = (mm >> 2) * EMBED
          segf = (mm & 3) * EMBED
          row = g * 16 + jj
          for c in range(EMBED // 16):
            t16 = tok_b[row, pl.ds(h64 + c * 16, 16)]
            pf = posf + c * 16
            p16 = pos_v[pf >> 7, pl.ds(pf & 127, 16)]
            sf = segf + c * 16
            s16 = seg_v[sf >> 7, pl.ds(sf & 127, 16)]
            out_b[row, pl.ds(c * 16, 16)] = t16 + p16 + s16

    g_start(0, tok0, gs0)

    def pair(i, carry):
      j = 2 * i
      g_start(j + 1, tok1, gs1)

      @pl.when(i > 0)
      def _():
        s_wait(j - 2, out0, ss0)
      g_wait(j, tok0, gs0)
      process(j, tok0, out0)

      @pl.when(i < CHUNKS_PER_W // 2 - 1)
      def _():
        g_start(j + 2, tok0, gs0)
      s_start(j, out0, ss0)

      @pl.when(i > 0)
      def _():
        s_wait(j - 1, out1, ss1)
      g_wait(j + 1, tok1, gs1)
      process(j + 1, tok1, out1)
      s_start(j + 1, out1, ss1)
      return carry
    lax.fori_loop(0, CHUNKS_PER_W // 2, pair, 0)
    s_wait(CHUNKS_PER_W - 2, out0, ss0)
    s_wait(CHUNKS_PER_W - 1, out1, ss1)

  return k(seq2, lab2, tok2, pos2, seg2)


def kernel(sequence, segment_labels, token_emb, position_emb, segment_emb):
  seq2 = sequence.reshape(N // CHUNK, CHUNK)
  lab2 = segment_labels.reshape(N // CHUNK, CHUNK)
  tok2 = token_emb.reshape(VOCAB // 2, 2 * EMBED)
  pos2 = position_emb.reshape(SEQ * EMBED // 128, 128)
  seg2 = jnp.pad(segment_emb.reshape(3 * EMBED), (0, 64)).reshape(2, 128)
  out = _sc_embed(seq2, lab2, tok2, pos2, seg2)
  return out.reshape(BATCH, SEQ, EMBED)


# speedup vs baseline: 1.4666x; 1.4666x over previous
import functools

import jax
import jax.numpy as jnp
from jax import lax
from jax.experimental import pallas as pl
from jax.experimental.pallas import tpu as pltpu
from jax.experimental.pallas import tpu_sc as plsc

VOCAB = 1000000
EMBED = 64
SEQ = 200
BATCH = 4096

NC, NS = 2, 16
NW = NC * NS
N = BATCH * SEQ
CHUNK = 128
ROWS_PER_W = N // NW
CHUNKS_PER_W = ROWS_PER_W // CHUNK


def _sc_embed(seq2, lab2, tok2, pos2, seg2):
  mesh = plsc.VectorSubcoreMesh(core_axis_name="c", subcore_axis_name="s")

  @functools.partial(
      pl.kernel,
      mesh=mesh,
      compiler_params=pltpu.CompilerParams(use_tc_tiling_on_sc=True,
                                           needs_layout_passes=False),
      out_type=jax.ShapeDtypeStruct((N, EMBED), jnp.float32),
      scratch_types=[
          pltpu.VMEM((CHUNKS_PER_W, CHUNK), jnp.int32),
          pltpu.VMEM((CHUNKS_PER_W, CHUNK), jnp.int32),
          pltpu.VMEM((SEQ * EMBED // 128, 128), jnp.float32),
          pltpu.VMEM((2, 128), jnp.float32),
          pltpu.VMEM((CHUNK, 128), jnp.float32),
          pltpu.VMEM((CHUNK, 128), jnp.float32),
          pltpu.VMEM((CHUNK, EMBED), jnp.float32),
          pltpu.VMEM((CHUNK, EMBED), jnp.float32),
          pltpu.SemaphoreType.DMA,
          pltpu.SemaphoreType.DMA,
          pltpu.SemaphoreType.DMA,
          pltpu.SemaphoreType.DMA,
      ],
  )
  def k(seq_hbm, lab_hbm, tok_hbm, pos_hbm, seg_hbm, out_hbm,
        idx_v, meta_v, pos_v, seg_v, tok0, tok1, out0, out1,
        gs0, gs1, ss0, ss1):
    wid = lax.axis_index("s") * NC + lax.axis_index("c")
    r0 = pl.multiple_of(wid * CHUNKS_PER_W, 8)
    flat0 = pl.multiple_of(wid * ROWS_PER_W, 1024)

    pltpu.sync_copy(seq_hbm.at[pl.ds(r0, CHUNKS_PER_W)], idx_v)
    pltpu.sync_copy(lab_hbm.at[pl.ds(r0, CHUNKS_PER_W)], meta_v)
    pltpu.sync_copy(pos_hbm, pos_v)
    pltpu.sync_copy(seg_hbm, seg_v)

    iota = lax.iota(jnp.int32, 16)

    @plsc.parallel_loop(0, CHUNKS_PER_W, unroll=2)
    def fuse(r):
      base = flat0 + r * CHUNK
      for i in range(CHUNK // 16):
        sl = pl.ds(i * 16, 16)
        pos = (base + i * 16 + iota) % SEQ
        v = idx_v[r, sl]
        meta_v[r, sl] = (pos * 4 + meta_v[r, sl]) * 2 + (v & 1)
        idx_v[r, sl] = v >> 1

    def g_start(j, tok_b, sem):
      pltpu.async_copy(tok_hbm.at[idx_v.at[j, pl.ds(0, CHUNK // 2)]],
                       tok_b.at[pl.ds(0, CHUNK // 2)], sem)
      pltpu.async_copy(tok_hbm.at[idx_v.at[j, pl.ds(CHUNK // 2, CHUNK // 2)]],
                       tok_b.at[pl.ds(CHUNK // 2, CHUNK // 2)], sem)

    def g_wait(j, tok_b, sem):
      pltpu.make_async_copy(tok_hbm.at[idx_v.at[j, pl.ds(0, CHUNK // 2)]],
                            tok_b.at[pl.ds(0, CHUNK // 2)], sem).wait()
      pltpu.make_async_copy(tok_hbm.at[idx_v.at[j, pl.ds(CHUNK // 2, CHUNK // 2)]],
                            tok_b.at[pl.ds(CHUNK // 2, CHUNK // 2)], sem).wait()

    def s_start(j, out_b, sem):
      off = pl.multiple_of(flat0 + j * CHUNK, 8)
      pltpu.async_copy(out_b, out_hbm.at[pl.ds(off, CHUNK)], sem)

    def s_wait(j, out_b, sem):
      off = pl.multiple_of(flat0 + j * CHUNK, 8)
      pltpu.make_async_copy(out_b, out_hbm.at[pl.ds(off, CHUNK)], sem).wait()

    def process(j, tok_b, out_b):
      @plsc.parallel_loop(0, CHUNK // 16, unroll=2)
      def group(g):
        mvec = meta_v[j, pl.ds(g * 16, 16)]
        for jj in range(16):
          m = mvec[jj]
          h64 = (m & 1) * EMBED
          mm = m >> 1
          posf = (mm >> 2) * EMBED
          segf = (mm & 3) * EMBED
          row = g * 16 + jj
          for c in range(EMBED // 16):
            t16 = tok_b[row, pl.ds(h64 + c * 16, 16)]
            pf = posf + c * 16
            p16 = pos_v[pf >> 7, pl.ds(pf & 127, 16)]
            sf = segf + c * 16
            s16 = seg_v[sf >> 7, pl.ds(sf & 127, 16)]
            out_b[row, pl.ds(c * 16, 16)] = t16 + p16 + s16

    g_start(0, tok0, gs0)

    def pair(i, carry):
      j = 2 * i
      g_start(j + 1, tok1, gs1)

      @pl.when(i > 0)
      def _():
        s_wait(j - 2, out0, ss0)
      g_wait(j, tok0, gs0)
      process(j, tok0, out0)

      @pl.when(i < CHUNKS_PER_W // 2 - 1)
      def _():
        g_start(j + 2, tok0, gs0)
      s_start(j, out0, ss0)

      @pl.when(i > 0)
      def _():
        s_wait(j - 1, out1, ss1)
      g_wait(j + 1, tok1, gs1)
      process(j + 1, tok1, out1)
      s_start(j + 1, out1, ss1)
      return carry
    lax.fori_loop(0, CHUNKS_PER_W // 2, pair, 0)
    s_wait(CHUNKS_PER_W - 2, out0, ss0)
    s_wait(CHUNKS_PER_W - 1, out1, ss1)

  return k(seq2, lab2, tok2, pos2, seg2)


def kernel(sequence, segment_labels, token_emb, position_emb, segment_emb):
  seq2 = sequence.reshape(N // CHUNK, CHUNK)
  lab2 = segment_labels.reshape(N // CHUNK, CHUNK)
  tok2 = token_emb.reshape(VOCAB // 2, 2 * EMBED)
  pos2 = position_emb.reshape(SEQ * EMBED // 128, 128)
  seg2 = jnp.pad(segment_emb.reshape(3 * EMBED), (0, 64)).reshape(2, 128)
  out = _sc_embed(seq2, lab2, tok2, pos2, seg2)
  return out.reshape(BATCH, SEQ, EMBED)
